# baseline (speedup 1.0000x reference)
import functools

import jax
import jax.numpy as jnp
from jax import lax
from jax.experimental import pallas as pl
from jax.experimental.pallas import tpu as pltpu
from jax.experimental.pallas import tpu_sc as plsc

B = 1024
L = 200
E = 128
Y = 50
YP = 64
YV = YP // 16

NC = 2
NS = 16
NW = NC * NS
BPW = B // NW
NLANE = 16
EV = E // NLANE
NBUF = 4

_mesh = plsc.VectorSubcoreMesh(core_axis_name="c", subcore_axis_name="s")


@functools.partial(
    pl.kernel,
    mesh=_mesh,
    out_type=jax.ShapeDtypeStruct((B, YP), jnp.float32),
    scratch_types=[
        pltpu.VMEM((BPW * L,), jnp.int32),
        pltpu.VMEM((NBUF, L, E), jnp.float32),
        pltpu.VMEM((E, YP), jnp.float32),
        pltpu.VMEM((YP,), jnp.float32),
        pltpu.VMEM((E,), jnp.float32),
        pltpu.VMEM((BPW, YP), jnp.float32),
    ] + [pltpu.SemaphoreType.DMA] * NBUF,
)
def _logreg_sc(x_hbm, w_hbm, fcwt_hbm, fcb_hbm, out_hbm,
               idx_v, bufs, fcw_v, fcb_v, prow, out_v, *sems):
    wid = lax.axis_index("s") * NC + lax.axis_index("c")
    base = wid * BPW

    pltpu.sync_copy(x_hbm.at[pl.ds(base * L, BPW * L)], idx_v)
    pltpu.sync_copy(fcwt_hbm, fcw_v)
    pltpu.sync_copy(fcb_hbm, fcb_v)

    def issue(r, b, sem):
        pltpu.async_copy(
            w_hbm.at[idx_v.at[pl.ds(r * L, 128)]],
            bufs.at[b, pl.ds(0, 128)], sem)
        pltpu.async_copy(
            w_hbm.at[idx_v.at[pl.ds(r * L + 128, L - 128)]],
            bufs.at[b, pl.ds(128, L - 128)], sem)

    def consume(r, b, sem):
        pltpu.make_async_copy(w_hbm.at[pl.ds(0, L)], bufs.at[b], sem).wait()

        def acc_body(j, accs):
            out = []
            for e, a in enumerate(accs):
                sl = pl.ds(e * NLANE, NLANE)
                s01 = bufs[b, 4 * j, sl] + bufs[b, 4 * j + 1, sl]
                s23 = bufs[b, 4 * j + 2, sl] + bufs[b, 4 * j + 3, sl]
                out.append(a + (s01 + s23))
            return tuple(out)

        accs = lax.fori_loop(
            0, L // 4, acc_body,
            tuple(jnp.zeros((NLANE,), jnp.float32) for _ in range(EV)))
        for e in range(EV):
            prow[pl.ds(e * NLANE, NLANE)] = accs[e]

        def dense_body(k, os):
            pv = prow[pl.ds(k * NLANE, NLANE)]
            os = list(os)
            for u in range(NLANE):
                pe = pv[u]
                e = k * NLANE + u
                for yc in range(YV):
                    os[yc] = os[yc] + pe * fcw_v[e, pl.ds(yc * NLANE, NLANE)]
            return tuple(os)

        ovecs = lax.fori_loop(
            0, EV, dense_body,
            tuple(fcb_v[pl.ds(yc * NLANE, NLANE)] for yc in range(YV)))
        for yc in range(YV):
            out_v[r, pl.ds(yc * NLANE, NLANE)] = ovecs[yc]

    for b in range(NBUF - 1):
        issue(b, b, sems[b])

    def grp_body(g, carry):
        for b in range(NBUF):
            r = g * NBUF + b
            nxt = r + NBUF - 1
            nb = (b + NBUF - 1) % NBUF

            @pl.when(nxt < BPW)
            def _():
                issue(nxt, nb, sems[nb])

            consume(r, b, sems[b])
        return carry

    lax.fori_loop(0, BPW // NBUF, grp_body, 0)
    pltpu.sync_copy(out_v, out_hbm.at[pl.ds(base, BPW)])


def kernel(x, W, fc_w, fc_b):
    xf = x.reshape(B * L).astype(jnp.int32)
    fcwt = jnp.pad(fc_w.T, ((0, 0), (0, YP - Y)))
    fcb = jnp.pad(fc_b, (0, YP - Y))
    out = _logreg_sc(xf, W, fcwt, fcb)
    return out[:, :Y]


# speedup vs baseline: 1.1856x; 1.1856x over previous
import functools

import jax
import jax.numpy as jnp
from jax import lax
from jax.experimental import pallas as pl
from jax.experimental.pallas import tpu as pltpu
from jax.experimental.pallas import tpu_sc as plsc

B = 1024
L = 200
E = 128
Y = 50

NC = 2
NS = 16
NW = NC * NS
BPW = B // NW
NLANE = 16
EV = E // NLANE
NBUF = 4

_mesh = plsc.VectorSubcoreMesh(core_axis_name="c", subcore_axis_name="s")


@functools.partial(
    pl.kernel,
    mesh=_mesh,
    out_type=jax.ShapeDtypeStruct((B, E), jnp.float32),
    scratch_types=[
        pltpu.VMEM((BPW * L,), jnp.int32),
        pltpu.VMEM((NBUF, L, E), jnp.float32),
        pltpu.VMEM((BPW, E), jnp.float32),
    ] + [pltpu.SemaphoreType.DMA] * NBUF,
)
def _pool_sc(x_hbm, w_hbm, out_hbm, idx_v, bufs, pooled_v, *sems):
    wid = lax.axis_index("s") * NC + lax.axis_index("c")
    base = wid * BPW

    pltpu.sync_copy(x_hbm.at[pl.ds(base * L, BPW * L)], idx_v)

    def issue(r, b, sem):
        pltpu.async_copy(
            w_hbm.at[idx_v.at[pl.ds(r * L, 128)]],
            bufs.at[b, pl.ds(0, 128)], sem)
        pltpu.async_copy(
            w_hbm.at[idx_v.at[pl.ds(r * L + 128, L - 128)]],
            bufs.at[b, pl.ds(128, L - 128)], sem)

    def consume(r, b, sem):
        pltpu.make_async_copy(w_hbm.at[pl.ds(0, L)], bufs.at[b], sem).wait()

        def acc_body(j, accs):
            out = []
            for e, a in enumerate(accs):
                sl = pl.ds(e * NLANE, NLANE)
                s01 = bufs[b, 4 * j, sl] + bufs[b, 4 * j + 1, sl]
                s23 = bufs[b, 4 * j + 2, sl] + bufs[b, 4 * j + 3, sl]
                out.append(a + (s01 + s23))
            return tuple(out)

        accs = lax.fori_loop(
            0, L // 4, acc_body,
            tuple(jnp.zeros((NLANE,), jnp.float32) for _ in range(EV)))
        for e in range(EV):
            pooled_v[r, pl.ds(e * NLANE, NLANE)] = accs[e]

    for b in range(NBUF - 1):
        issue(b, b, sems[b])

    def grp_body(g, carry):
        for b in range(NBUF):
            r = g * NBUF + b
            nxt = r + NBUF - 1
            nb = (b + NBUF - 1) % NBUF

            @pl.when(nxt < BPW)
            def _():
                issue(nxt, nb, sems[nb])

            consume(r, b, sems[b])
        return carry

    lax.fori_loop(0, BPW // NBUF, grp_body, 0)
    pltpu.sync_copy(pooled_v, out_hbm.at[pl.ds(base, BPW)])


def _dense_tc(p_ref, w_ref, b_ref, o_ref):
    o_ref[...] = lax.dot_general(
        p_ref[...], w_ref[...], (((1,), (1,)), ((), ())),
        preferred_element_type=jnp.float32) + b_ref[...]


def kernel(x, W, fc_w, fc_b):
    xf = x.reshape(B * L).astype(jnp.int32)
    pooled = _pool_sc(xf, W)
    out = pl.pallas_call(
        _dense_tc,
        out_shape=jax.ShapeDtypeStruct((B, Y), jnp.float32),
    )(pooled, fc_w, fc_b.reshape(1, Y))
    return out
